# baseline (speedup 1.0000x reference)
import jax
import jax.numpy as jnp
from jax import lax
from jax.experimental import pallas as pl
from jax.experimental.pallas import tpu as pltpu
from jax.experimental.pallas import tpu_sc as plsc

B, H, W, G, C, K = 4, 56, 56, 8, 16, 9
NW = 32
WP = 64
NCHUNK = WP // 16
PAR_ROW = 3 * K * WP
PAR_PAD = 1792
OUT_ROW = C * W
INP_WORDS = H * W * C
PAR_OFF = INP_WORDS


def _dcn_body(inp_hbm, par_hbm, out_hbm, slab, par_v, out_v,
              sem_in, sem_par, sem_out):
    cid = lax.axis_index("c")
    sid = lax.axis_index("s")
    wid = sid * 2 + cid

    pltpu.sync_copy(inp_hbm.at[wid], slab)

    pltpu.async_copy(par_hbm.at[wid, 0], par_v.at[pl.ds(0, PAR_PAD)], sem_par)

    iota = lax.iota(jnp.int32, 16)
    iota_f = iota.astype(jnp.float32)

    def row_step(h, _):
        buf = lax.rem(h, 2)
        pbase = buf * PAR_PAD
        obase = buf * OUT_ROW

        @pl.when(h + 1 < H)
        def _start_next_par():
            nbase = lax.rem(h + 1, 2) * PAR_PAD
            pltpu.async_copy(par_hbm.at[wid, h + 1],
                             par_v.at[pl.ds(nbase, PAR_PAD)], sem_par)

        pltpu.make_async_copy(par_hbm.at[wid, h],
                              par_v.at[pl.ds(pbase, PAR_PAD)], sem_par).wait()

        @pl.when(h >= 2)
        def _drain_out():
            pltpu.make_async_copy(out_v.at[pl.ds(obase, OUT_ROW)],
                                  out_hbm.at[wid, h - 2], sem_out).wait()

        hf = h.astype(jnp.float32)

        @plsc.parallel_loop(0, NCHUNK)
        def chunk_body(cb):
            cb16 = cb * 16
            wvec = iota_f + cb16.astype(jnp.float32)
            accs = tuple(jnp.zeros((16,), jnp.float32) for _ in range(C))

            for k in range(K):
                pidx = pbase + (k * WP + cb16) + iota
                dxv = plsc.load_gather(par_v, [pidx])
                dyv = plsc.load_gather(par_v, [pidx + K * WP])
                wkv = plsc.load_gather(par_v, [pidx + 2 * K * WP])
                x = dxv + wvec
                y = dyv + hf
                xi = jnp.clip(x, -4.0, 60.0).astype(jnp.int32)
                yi = jnp.clip(y, -4.0, 60.0).astype(jnp.int32)
                fx = xi.astype(jnp.float32)
                fy = yi.astype(jnp.float32)
                tx = x - fx
                ty = y - fy
                ox = 1.0 - tx
                oy = 1.0 - ty
                mx0 = (xi >= 0) & (xi < W)
                mx1 = (xi >= -1) & (xi < W - 1)
                my0 = (yi >= 0) & (yi < H)
                my1 = (yi >= -1) & (yi < H - 1)
                zero = jnp.zeros((16,), jnp.float32)
                wtl = jnp.where(mx0 & my0, wkv * (ox * oy), zero)
                wtr = jnp.where(mx1 & my0, wkv * (tx * oy), zero)
                wbl = jnp.where(mx0 & my1, wkv * (ox * ty), zero)
                wbr = jnp.where(mx1 & my1, wkv * (tx * ty), zero)
                x0 = jnp.clip(xi, 0, W - 1)
                x1 = jnp.clip(xi + 1, 0, W - 1)
                y0 = jnp.clip(yi, 0, H - 1) * W
                y1 = jnp.clip(yi + 1, 0, H - 1) * W
                a00 = y0 + x0
                a10 = y0 + x1
                a01 = y1 + x0
                a11 = y1 + x1
                new = []
                for c in range(C):
                    v00 = plsc.load_gather(slab, [a00 + c * (H * W)])
                    v10 = plsc.load_gather(slab, [a10 + c * (H * W)])
                    v01 = plsc.load_gather(slab, [a01 + c * (H * W)])
                    v11 = plsc.load_gather(slab, [a11 + c * (H * W)])
                    new.append(accs[c] + (v00 * wtl + v10 * wtr
                                          + v01 * wbl + v11 * wbr))
                accs = tuple(new)

            oidx = obase + cb16 + iota
            omask = (cb16 + iota) < W
            for c in range(C):
                plsc.store_scatter(out_v, [oidx + c * W], accs[c], mask=omask)

        pltpu.async_copy(out_v.at[pl.ds(obase, OUT_ROW)],
                         out_hbm.at[wid, h], sem_out)
        return 0

    lax.fori_loop(0, H, row_step, 0)

    pltpu.make_async_copy(out_v.at[pl.ds(0, OUT_ROW)],
                          out_hbm.at[wid, H - 2], sem_out).wait()
    pltpu.make_async_copy(out_v.at[pl.ds(OUT_ROW, OUT_ROW)],
                          out_hbm.at[wid, H - 1], sem_out).wait()


@jax.jit
def kernel(input, deformable, weights):
    inp_t = input.transpose(0, 3, 4, 1, 2).reshape(NW, INP_WORDS)
    dx = deformable[..., 0].transpose(0, 3, 1, 4, 2)
    dy = deformable[..., 1].transpose(0, 3, 1, 4, 2)
    wk = weights.transpose(0, 3, 1, 4, 2)
    par = jnp.concatenate([dx, dy, wk], axis=3)
    par = jnp.pad(par, ((0, 0), (0, 0), (0, 0), (0, 0), (0, WP - W)))
    par = par.reshape(NW, H, PAR_ROW)
    par = jnp.pad(par, ((0, 0), (0, 0), (0, PAR_PAD - PAR_ROW)))


    mesh = plsc.VectorSubcoreMesh(core_axis_name="c", subcore_axis_name="s",
                                  num_cores=2, num_subcores=16)
    run = pl.kernel(
        _dcn_body,
        out_type=jax.ShapeDtypeStruct((NW, H, OUT_ROW), jnp.float32),
        mesh=mesh,
        scratch_types=[
            pltpu.VMEM((H * W * C,), jnp.float32),
            pltpu.VMEM((2 * PAR_PAD,), jnp.float32),
            pltpu.VMEM((2 * OUT_ROW,), jnp.float32),
            pltpu.SemaphoreType.DMA,
            pltpu.SemaphoreType.DMA,
            pltpu.SemaphoreType.DMA,
        ],
        compiler_params=pltpu.CompilerParams(needs_layout_passes=False),
    )
    out = run(inp_t, par)
    out = out.reshape(B, G, H, C, W)
    return out.transpose(0, 2, 4, 1, 3)
